# baseline (speedup 1.0000x reference)
import functools

import jax
import jax.numpy as jnp
from jax import lax
from jax.experimental import pallas as pl
from jax.experimental.pallas import tpu as pltpu
from jax.experimental.pallas import tpu_sc as plsc

_N = 50000
_C = 128
_COUT = 128
_K = 9
_KSPLIT = 4
_MAX_DEPTH = 8

_BN = 256
_NBLK_TC = (_N + _BN - 1) // _BN + 1
_NPAD = _BN * _NBLK_TC
_B = 128
_NBLK_SC = _NPAD // _B
_NW = 32
_TMAX = (_NBLK_SC + _NW - 1) // _NW
_ZROW = _N


def _make_tc(k_slots, with_base):
    kg = len(k_slots)

    def body(*refs):
        if with_base:
            x_ref, deltaT_ref, w_ref, b_ref, y_ref, base_ref = refs
        else:
            x_ref, w_ref, y_ref = refs
        i = pl.program_id(0)
        row = i * _BN + lax.broadcasted_iota(jnp.int32, (_BN, 1), 0)
        xb = jnp.where(row < _N, x_ref[...], 0.0)
        for g, k in enumerate(k_slots):
            wk = w_ref[k * (_C + 1):k * (_C + 1) + _C, :]
            y_ref[g, :, :] = jnp.dot(xb, wk, preferred_element_type=jnp.float32)
        if with_base:
            deltaT = deltaT_ref[...]
            wd = jnp.concatenate(
                [w_ref[k * (_C + 1) + _C:k * (_C + 1) + _C + 1, :]
                 for k in range(_K)], axis=0)
            base = lax.dot_general(deltaT, wd, (((0,), (0,)), ((), ())),
                                   preferred_element_type=jnp.float32)
            base_ref[...] = base + b_ref[...]

    in_specs = [pl.BlockSpec((_BN, _C), lambda i: (jnp.minimum(i, _N // _BN), 0))]
    if with_base:
        in_specs.append(pl.BlockSpec((_K, _BN), lambda i: (0, i)))
    in_specs.append(pl.BlockSpec((_K * (_C + 1), _COUT), lambda i: (0, 0)))
    out_specs = [pl.BlockSpec((kg, _BN, _C), lambda i: (0, i, 0))]
    out_shape = [jax.ShapeDtypeStruct((kg, _NPAD, _C), jnp.float32)]
    if with_base:
        in_specs.append(pl.BlockSpec((1, _COUT), lambda i: (0, 0)))
        out_specs.append(pl.BlockSpec((_BN, _COUT), lambda i: (i, 0)))
        out_shape.append(jax.ShapeDtypeStruct((_NPAD, _COUT), jnp.float32))

    return pl.pallas_call(
        body,
        grid=(_NBLK_TC,),
        in_specs=in_specs,
        out_specs=out_specs,
        out_shape=out_shape,
    )


_tc_a = _make_tc(tuple(range(_KSPLIT)), with_base=True)
_tc_b = _make_tc(tuple(range(_KSPLIT, _K)), with_base=False)


def _make_sc(kg):

    @functools.partial(
        pl.kernel,
        out_type=jax.ShapeDtypeStruct((_NPAD, _COUT), jnp.float32),
        mesh=plsc.VectorSubcoreMesh(core_axis_name="c", subcore_axis_name="s"),
        scratch_types=[
            pltpu.VMEM((kg, _B), jnp.int32),
            pltpu.VMEM((kg, _B), jnp.int32),
            pltpu.VMEM((_B, _COUT), jnp.float32),
            pltpu.SemaphoreType.DMA,
        ],
        compiler_params=pltpu.CompilerParams(use_tc_tiling_on_sc=True),
    )
    def sc_gather_sum(ytab, base_hbm, idxb_hbm, out_hbm, idxv, jv, acc, sem):
        cid = lax.axis_index("c")
        sid = lax.axis_index("s")
        wid = sid * 2 + cid

        def block_body(t, carry):
            blk = wid + _NW * t

            @pl.when(blk < _NBLK_SC)
            def _():
                off = blk * _B
                pltpu.sync_copy(idxb_hbm.at[blk], idxv)
                pltpu.sync_copy(base_hbm.at[pl.ds(off, _B)], acc)
                for k in range(kg):
                    for u in range(_B // 16):
                        v = idxv[k, pl.ds(u * 16, 16)]
                        zrow = _ZROW + u * 16 + lax.iota(jnp.int32, 16)
                        jv[k, pl.ds(u * 16, 16)] = jnp.where(
                            v >= 0, v + k * _NPAD, zrow + k * _NPAD)
                cps = [
                    pltpu.async_copy(ytab.at[jv.at[k]], acc, sem, add=True)
                    for k in range(kg)
                ]
                for cp in cps:
                    cp.wait()
                pltpu.sync_copy(acc, out_hbm.at[pl.ds(off, _B)])

            return carry

        lax.fori_loop(0, _TMAX, block_body, 0)

    return sc_gather_sum


_sc_a = _make_sc(_KSPLIT)
_sc_b = _make_sc(_K - _KSPLIT)


def kernel(x, neigh_idx, neigh_depth, depth, W, b):
    idxT = jnp.pad(neigh_idx.astype(jnp.int32).T, ((0, 0), (0, _NPAD - _N)),
                   constant_values=-1)
    valid = (idxT >= 0).astype(jnp.float32)
    ndT = jnp.pad(neigh_depth.astype(jnp.float32).T,
                  ((0, 0), (0, _NPAD - _N)))
    deltaT = (jnp.asarray(depth, jnp.float32) - ndT) * (1.0 / _MAX_DEPTH) * valid
    b2 = b.reshape(1, _COUT).astype(jnp.float32)

    idxb = idxT.reshape(_K, _NBLK_SC, _B).transpose(1, 0, 2)
    xf = x.astype(jnp.float32)
    Wf = W.astype(jnp.float32)

    ya, base = _tc_a(xf, deltaT, Wf, b2)
    (yb,) = _tc_b(xf, Wf)

    part = _sc_a(ya.reshape(_KSPLIT * _NPAD, _C), base,
                 idxb[:, :_KSPLIT])
    out = _sc_b(yb.reshape((_K - _KSPLIT) * _NPAD, _C), part,
                idxb[:, _KSPLIT:])
    return out[:_N]


# speedup vs baseline: 1.2757x; 1.2757x over previous
import functools

import jax
import jax.numpy as jnp
from jax import lax
from jax.experimental import pallas as pl
from jax.experimental.pallas import tpu as pltpu
from jax.experimental.pallas import tpu_sc as plsc

_N = 50000
_C = 128
_COUT = 128
_K = 9
_MAX_DEPTH = 8

_BN = 256
_BN_LOG2 = 8
_NBLK_TC = (_N + _BN - 1) // _BN + 1
_NPAD = _BN * _NBLK_TC
_B = 128
_NBLK_SC = _NPAD // _B
_NW = 32
_TMAX = (_NBLK_SC + _NW - 1) // _NW
_ZROW = _N


def _tc_body(x_ref, deltaT_ref, w_ref, b_ref, y_ref, base_ref):
    i = pl.program_id(0)
    row = i * _BN + lax.broadcasted_iota(jnp.int32, (_BN, 1), 0)
    xb = jnp.where(row < _N, x_ref[...], 0.0)
    for k in range(_K):
        wk = w_ref[k * (_C + 1):k * (_C + 1) + _C, :]
        y_ref[0, k * _BN:(k + 1) * _BN, :] = jnp.dot(
            xb, wk, preferred_element_type=jnp.float32)
    deltaT = deltaT_ref[...]
    wd = jnp.concatenate(
        [w_ref[k * (_C + 1) + _C:k * (_C + 1) + _C + 1, :] for k in range(_K)],
        axis=0)
    base = lax.dot_general(deltaT, wd, (((0,), (0,)), ((), ())),
                           preferred_element_type=jnp.float32)
    base_ref[...] = base + b_ref[...]


def _tc_phase(x, deltaT, W, b2):
    return pl.pallas_call(
        _tc_body,
        grid=(_NBLK_TC,),
        in_specs=[
            pl.BlockSpec((_BN, _C), lambda i: (jnp.minimum(i, _N // _BN), 0)),
            pl.BlockSpec((_K, _BN), lambda i: (0, i)),
            pl.BlockSpec((_K * (_C + 1), _COUT), lambda i: (0, 0)),
            pl.BlockSpec((1, _COUT), lambda i: (0, 0)),
        ],
        out_specs=[
            pl.BlockSpec((1, _K * _BN, _C), lambda i: (i, 0, 0)),
            pl.BlockSpec((_BN, _COUT), lambda i: (i, 0)),
        ],
        out_shape=[
            jax.ShapeDtypeStruct((_NBLK_TC, _K * _BN, _C), jnp.float32),
            jax.ShapeDtypeStruct((_NPAD, _COUT), jnp.float32),
        ],
    )(x, deltaT, W, b2)


@functools.partial(
    pl.kernel,
    out_type=jax.ShapeDtypeStruct((_NPAD, _COUT), jnp.float32),
    mesh=plsc.VectorSubcoreMesh(core_axis_name="c", subcore_axis_name="s"),
    scratch_types=[
        pltpu.VMEM((_K, _B), jnp.int32),
        pltpu.VMEM((_K, _B), jnp.int32),
        pltpu.VMEM((_B, _COUT), jnp.float32),
        pltpu.SemaphoreType.DMA,
    ],
    compiler_params=pltpu.CompilerParams(use_tc_tiling_on_sc=True),
)
def _sc_gather_sum(ytab, base_hbm, idxb_hbm, out_hbm, idxv, jv, acc, sem):
    cid = lax.axis_index("c")
    sid = lax.axis_index("s")
    wid = sid * 2 + cid

    def block_body(t, carry):
        blk = wid + _NW * t

        @pl.when(blk < _NBLK_SC)
        def _():
            off = blk * _B
            pltpu.sync_copy(idxb_hbm.at[blk], idxv)
            pltpu.sync_copy(base_hbm.at[pl.ds(off, _B)], acc)
            for k in range(_K):
                for u in range(_B // 16):
                    v = idxv[k, pl.ds(u * 16, 16)]
                    zrow = _ZROW + u * 16 + lax.iota(jnp.int32, 16)
                    vv = jnp.where(v >= 0, v, zrow)
                    jv[k, pl.ds(u * 16, 16)] = (
                        lax.shift_right_logical(vv, _BN_LOG2) * (_K * _BN)
                        + k * _BN + lax.bitwise_and(vv, _BN - 1))
            cps = [
                pltpu.async_copy(ytab.at[jv.at[k]], acc, sem, add=True)
                for k in range(_K)
            ]
            for cp in cps:
                cp.wait()
            pltpu.sync_copy(acc, out_hbm.at[pl.ds(off, _B)])

        return carry

    lax.fori_loop(0, _TMAX, block_body, 0)


def kernel(x, neigh_idx, neigh_depth, depth, W, b):
    idxT = jnp.pad(neigh_idx.astype(jnp.int32).T, ((0, 0), (0, _NPAD - _N)),
                   constant_values=-1)
    valid = (idxT >= 0).astype(jnp.float32)
    ndT = jnp.pad(neigh_depth.astype(jnp.float32).T,
                  ((0, 0), (0, _NPAD - _N)))
    deltaT = (jnp.asarray(depth, jnp.float32) - ndT) * (1.0 / _MAX_DEPTH) * valid
    b2 = b.reshape(1, _COUT).astype(jnp.float32)
    idxb = idxT.reshape(_K, _NBLK_SC, _B).transpose(1, 0, 2)

    y, base = _tc_phase(x.astype(jnp.float32), deltaT, W.astype(jnp.float32),
                        b2)
    ytab = y.reshape(_NBLK_TC * _K * _BN, _C)

    out = _sc_gather_sum(ytab, base, idxb)
    return out[:_N]


# speedup vs baseline: 1.4947x; 1.1717x over previous
import functools

import jax
import jax.numpy as jnp
from jax import lax
from jax.experimental import pallas as pl
from jax.experimental.pallas import tpu as pltpu
from jax.experimental.pallas import tpu_sc as plsc

_N = 50000
_C = 128
_COUT = 128
_K = 9
_MAX_DEPTH = 8

_BN = 512
_BN_LOG2 = 9
_NBLK_TC = (_N + _BN - 1) // _BN + 1
_NPAD = _BN * _NBLK_TC
_B = 128
_NBLK_SC = _NPAD // _B
_NW = 32
_TMAX = (_NBLK_SC + _NW - 1) // _NW
_ZROW = _N


def _tc_body(x_ref, deltaT_ref, w_ref, b_ref, y_ref, base_ref):
    i = pl.program_id(0)
    row = i * _BN + lax.broadcasted_iota(jnp.int32, (_BN, 1), 0)
    xb = jnp.where(row < _N, x_ref[...], 0.0)
    for k in range(_K):
        wk = w_ref[k * (_C + 1):k * (_C + 1) + _C, :]
        y_ref[0, k * _BN:(k + 1) * _BN, :] = jnp.dot(
            xb, wk, preferred_element_type=jnp.float32)
    deltaT = deltaT_ref[...]
    wd = jnp.concatenate(
        [w_ref[k * (_C + 1) + _C:k * (_C + 1) + _C + 1, :] for k in range(_K)],
        axis=0)
    base = lax.dot_general(deltaT, wd, (((0,), (0,)), ((), ())),
                           preferred_element_type=jnp.float32)
    base_ref[...] = base + b_ref[...]


def _tc_phase(x, deltaT, W, b2):
    return pl.pallas_call(
        _tc_body,
        grid=(_NBLK_TC,),
        in_specs=[
            pl.BlockSpec((_BN, _C), lambda i: (jnp.minimum(i, _N // _BN), 0)),
            pl.BlockSpec((_K, _BN), lambda i: (0, i)),
            pl.BlockSpec((_K * (_C + 1), _COUT), lambda i: (0, 0)),
            pl.BlockSpec((1, _COUT), lambda i: (0, 0)),
        ],
        out_specs=[
            pl.BlockSpec((1, _K * _BN, _C), lambda i: (i, 0, 0)),
            pl.BlockSpec((_BN, _COUT), lambda i: (i, 0)),
        ],
        out_shape=[
            jax.ShapeDtypeStruct((_NBLK_TC, _K * _BN, _C), jnp.float32),
            jax.ShapeDtypeStruct((_NPAD, _COUT), jnp.float32),
        ],
    )(x, deltaT, W, b2)


@functools.partial(
    pl.kernel,
    out_type=jax.ShapeDtypeStruct((_NPAD, _COUT), jnp.float32),
    mesh=plsc.VectorSubcoreMesh(core_axis_name="c", subcore_axis_name="s"),
    scratch_types=[
        pltpu.VMEM((_K, _B), jnp.int32),
        pltpu.VMEM((_K, _B), jnp.int32),
        pltpu.VMEM((_B, _COUT), jnp.float32),
        pltpu.SemaphoreType.DMA,
    ],
    compiler_params=pltpu.CompilerParams(use_tc_tiling_on_sc=True),
)
def _sc_gather_sum(ytab, base_hbm, idxb_hbm, out_hbm, idxv, jv, acc, sem):
    cid = lax.axis_index("c")
    sid = lax.axis_index("s")
    wid = sid * 2 + cid

    def block_body(t, carry):
        blk = wid + _NW * t

        @pl.when(blk < _NBLK_SC)
        def _():
            off = blk * _B
            pltpu.sync_copy(idxb_hbm.at[blk], idxv)
            pltpu.sync_copy(base_hbm.at[pl.ds(off, _B)], acc)
            for k in range(_K):
                for u in range(_B // 16):
                    v = idxv[k, pl.ds(u * 16, 16)]
                    zrow = _ZROW + u * 16 + lax.iota(jnp.int32, 16)
                    vv = jnp.where(v >= 0, v, zrow)
                    jv[k, pl.ds(u * 16, 16)] = (
                        lax.shift_right_logical(vv, _BN_LOG2) * (_K * _BN)
                        + k * _BN + lax.bitwise_and(vv, _BN - 1))
            cps = [
                pltpu.async_copy(ytab.at[jv.at[k]], acc, sem, add=True)
                for k in range(_K)
            ]
            for cp in cps:
                cp.wait()
            pltpu.sync_copy(acc, out_hbm.at[pl.ds(off, _B)])

        return carry

    lax.fori_loop(0, _TMAX, block_body, 0)


def kernel(x, neigh_idx, neigh_depth, depth, W, b):
    idxT = jnp.pad(neigh_idx.astype(jnp.int32).T, ((0, 0), (0, _NPAD - _N)),
                   constant_values=-1)
    valid = (idxT >= 0).astype(jnp.float32)
    ndT = jnp.pad(neigh_depth.astype(jnp.float32).T,
                  ((0, 0), (0, _NPAD - _N)))
    deltaT = (jnp.asarray(depth, jnp.float32) - ndT) * (1.0 / _MAX_DEPTH) * valid
    b2 = b.reshape(1, _COUT).astype(jnp.float32)
    idxb = idxT.reshape(_K, _NBLK_SC, _B).transpose(1, 0, 2)

    y, base = _tc_phase(x.astype(jnp.float32), deltaT, W.astype(jnp.float32),
                        b2)
    ytab = y.reshape(_NBLK_TC * _K * _BN, _C)

    out = _sc_gather_sum(ytab, base, idxb)
    return out[:_N]


# speedup vs baseline: 1.6735x; 1.1196x over previous
import functools

import jax
import jax.numpy as jnp
from jax import lax
from jax.experimental import pallas as pl
from jax.experimental.pallas import tpu as pltpu
from jax.experimental.pallas import tpu_sc as plsc

_N = 50000
_C = 128
_COUT = 128
_K = 9
_MAX_DEPTH = 8

_BN = 1024
_BN_LOG2 = 10
_NBLK_TC = (_N + _BN - 1) // _BN + 1
_NPAD = _BN * _NBLK_TC
_B = 128
_NBLK_SC = _NPAD // _B
_NW = 32
_TMAX = (_NBLK_SC + _NW - 1) // _NW
_ZROW = _N


def _tc_body(x_ref, deltaT_ref, w_ref, b_ref, y_ref, base_ref):
    i = pl.program_id(0)
    row = i * _BN + lax.broadcasted_iota(jnp.int32, (_BN, 1), 0)
    xb = jnp.where(row < _N, x_ref[...], 0.0)
    for k in range(_K):
        wk = w_ref[k * (_C + 1):k * (_C + 1) + _C, :]
        y_ref[0, k * _BN:(k + 1) * _BN, :] = jnp.dot(
            xb, wk, preferred_element_type=jnp.float32)
    deltaT = deltaT_ref[...]
    wd = jnp.concatenate(
        [w_ref[k * (_C + 1) + _C:k * (_C + 1) + _C + 1, :] for k in range(_K)],
        axis=0)
    base = lax.dot_general(deltaT, wd, (((0,), (0,)), ((), ())),
                           preferred_element_type=jnp.float32)
    base_ref[...] = base + b_ref[...]


def _tc_phase(x, deltaT, W, b2):
    return pl.pallas_call(
        _tc_body,
        grid=(_NBLK_TC,),
        in_specs=[
            pl.BlockSpec((_BN, _C), lambda i: (jnp.minimum(i, _N // _BN), 0)),
            pl.BlockSpec((_K, _BN), lambda i: (0, i)),
            pl.BlockSpec((_K * (_C + 1), _COUT), lambda i: (0, 0)),
            pl.BlockSpec((1, _COUT), lambda i: (0, 0)),
        ],
        out_specs=[
            pl.BlockSpec((1, _K * _BN, _C), lambda i: (i, 0, 0)),
            pl.BlockSpec((_BN, _COUT), lambda i: (i, 0)),
        ],
        out_shape=[
            jax.ShapeDtypeStruct((_NBLK_TC, _K * _BN, _C), jnp.float32),
            jax.ShapeDtypeStruct((_NPAD, _COUT), jnp.float32),
        ],
    )(x, deltaT, W, b2)


@functools.partial(
    pl.kernel,
    out_type=jax.ShapeDtypeStruct((_NPAD, _COUT), jnp.float32),
    mesh=plsc.VectorSubcoreMesh(core_axis_name="c", subcore_axis_name="s"),
    scratch_types=[
        pltpu.VMEM((_K, _B), jnp.int32),
        pltpu.VMEM((_K, _B), jnp.int32),
        pltpu.VMEM((_B, _COUT), jnp.float32),
        pltpu.SemaphoreType.DMA,
    ],
    compiler_params=pltpu.CompilerParams(use_tc_tiling_on_sc=True),
)
def _sc_gather_sum(ytab, base_hbm, idxb_hbm, out_hbm, idxv, jv, acc, sem):
    cid = lax.axis_index("c")
    sid = lax.axis_index("s")
    wid = sid * 2 + cid

    def block_body(t, carry):
        blk = wid + _NW * t

        @pl.when(blk < _NBLK_SC)
        def _():
            off = blk * _B
            pltpu.sync_copy(idxb_hbm.at[blk], idxv)
            pltpu.sync_copy(base_hbm.at[pl.ds(off, _B)], acc)
            for k in range(_K):
                for u in range(_B // 16):
                    v = idxv[k, pl.ds(u * 16, 16)]
                    zrow = _ZROW + u * 16 + lax.iota(jnp.int32, 16)
                    vv = jnp.where(v >= 0, v, zrow)
                    jv[k, pl.ds(u * 16, 16)] = (
                        lax.shift_right_logical(vv, _BN_LOG2) * (_K * _BN)
                        + k * _BN + lax.bitwise_and(vv, _BN - 1))
            cps = [
                pltpu.async_copy(ytab.at[jv.at[k]], acc, sem, add=True)
                for k in range(_K)
            ]
            for cp in cps:
                cp.wait()
            pltpu.sync_copy(acc, out_hbm.at[pl.ds(off, _B)])

        return carry

    lax.fori_loop(0, _TMAX, block_body, 0)


def kernel(x, neigh_idx, neigh_depth, depth, W, b):
    idxT = jnp.pad(neigh_idx.astype(jnp.int32).T, ((0, 0), (0, _NPAD - _N)),
                   constant_values=-1)
    valid = (idxT >= 0).astype(jnp.float32)
    ndT = jnp.pad(neigh_depth.astype(jnp.float32).T,
                  ((0, 0), (0, _NPAD - _N)))
    deltaT = (jnp.asarray(depth, jnp.float32) - ndT) * (1.0 / _MAX_DEPTH) * valid
    b2 = b.reshape(1, _COUT).astype(jnp.float32)
    idxb = idxT.reshape(_K, _NBLK_SC, _B).transpose(1, 0, 2)

    y, base = _tc_phase(x.astype(jnp.float32), deltaT, W.astype(jnp.float32),
                        b2)
    ytab = y.reshape(_NBLK_TC * _K * _BN, _C)

    out = _sc_gather_sum(ytab, base, idxb)
    return out[:_N]


# speedup vs baseline: 1.6858x; 1.0074x over previous
import functools

import jax
import jax.numpy as jnp
from jax import lax
from jax.experimental import pallas as pl
from jax.experimental.pallas import tpu as pltpu
from jax.experimental.pallas import tpu_sc as plsc

_N = 50000
_C = 128
_COUT = 128
_K = 9
_MAX_DEPTH = 8

_BN = 1024
_BN_LOG2 = 10
_NBLK_TC = (_N + _BN - 1) // _BN
_NPAD = _BN * _NBLK_TC
_B = 128
_NBLK_SC = _NPAD // _B
_NW = 32
_TMAX = (_NBLK_SC + _NW - 1) // _NW
_ZROW = _N


def _tc_body(x_ref, deltaT_ref, w_ref, b_ref, y_ref, base_ref):
    i = pl.program_id(0)
    row = i * _BN + lax.broadcasted_iota(jnp.int32, (_BN, 1), 0)
    xb = jnp.where(row < _N, x_ref[...], 0.0)
    for k in range(_K):
        wk = w_ref[k * (_C + 1):k * (_C + 1) + _C, :]
        y_ref[0, k * _BN:(k + 1) * _BN, :] = jnp.dot(
            xb, wk, preferred_element_type=jnp.float32)
    deltaT = deltaT_ref[...]
    wd = jnp.concatenate(
        [w_ref[k * (_C + 1) + _C:k * (_C + 1) + _C + 1, :] for k in range(_K)],
        axis=0)
    base = lax.dot_general(deltaT, wd, (((0,), (0,)), ((), ())),
                           preferred_element_type=jnp.float32)
    base_ref[...] = base + b_ref[...]


def _tc_phase(x, deltaT, W, b2):
    return pl.pallas_call(
        _tc_body,
        grid=(_NBLK_TC,),
        in_specs=[
            pl.BlockSpec((_BN, _C), lambda i: (jnp.minimum(i, _N // _BN), 0)),
            pl.BlockSpec((_K, _BN), lambda i: (0, i)),
            pl.BlockSpec((_K * (_C + 1), _COUT), lambda i: (0, 0)),
            pl.BlockSpec((1, _COUT), lambda i: (0, 0)),
        ],
        out_specs=[
            pl.BlockSpec((1, _K * _BN, _C), lambda i: (i, 0, 0)),
            pl.BlockSpec((_BN, _COUT), lambda i: (i, 0)),
        ],
        out_shape=[
            jax.ShapeDtypeStruct((_NBLK_TC, _K * _BN, _C), jnp.float32),
            jax.ShapeDtypeStruct((_NPAD, _COUT), jnp.float32),
        ],
    )(x, deltaT, W, b2)


@functools.partial(
    pl.kernel,
    out_type=jax.ShapeDtypeStruct((_NPAD, _COUT), jnp.float32),
    mesh=plsc.VectorSubcoreMesh(core_axis_name="c", subcore_axis_name="s"),
    scratch_types=[
        pltpu.VMEM((_K, _B), jnp.int32),
        pltpu.VMEM((_K, _B), jnp.int32),
        pltpu.VMEM((_B, _COUT), jnp.float32),
        pltpu.SemaphoreType.DMA,
    ],
    compiler_params=pltpu.CompilerParams(use_tc_tiling_on_sc=True),
)
def _sc_gather_sum(ytab, base_hbm, idxb_hbm, out_hbm, idxv, jv, acc, sem):
    cid = lax.axis_index("c")
    sid = lax.axis_index("s")
    wid = sid * 2 + cid

    def block_body(t, carry):
        blk = wid + _NW * t

        @pl.when(blk < _NBLK_SC)
        def _():
            off = blk * _B
            pltpu.sync_copy(idxb_hbm.at[blk], idxv)
            pltpu.sync_copy(base_hbm.at[pl.ds(off, _B)], acc)
            for k in range(_K):
                for u in range(_B // 16):
                    v = idxv[k, pl.ds(u * 16, 16)]
                    zrow = _ZROW + u * 16 + lax.iota(jnp.int32, 16)
                    vv = jnp.where(v >= 0, v, zrow)
                    jv[k, pl.ds(u * 16, 16)] = (
                        lax.shift_right_logical(vv, _BN_LOG2) * (_K * _BN)
                        + k * _BN + lax.bitwise_and(vv, _BN - 1))
            cps = [
                pltpu.async_copy(ytab.at[jv.at[k]], acc, sem, add=True)
                for k in range(_K)
            ]
            for cp in cps:
                cp.wait()
            pltpu.sync_copy(acc, out_hbm.at[pl.ds(off, _B)])

        return carry

    lax.fori_loop(0, _TMAX, block_body, 0)


def kernel(x, neigh_idx, neigh_depth, depth, W, b):
    idxT = jnp.pad(neigh_idx.astype(jnp.int32).T, ((0, 0), (0, _NPAD - _N)),
                   constant_values=-1)
    valid = (idxT >= 0).astype(jnp.float32)
    ndT = jnp.pad(neigh_depth.astype(jnp.float32).T,
                  ((0, 0), (0, _NPAD - _N)))
    deltaT = (jnp.asarray(depth, jnp.float32) - ndT) * (1.0 / _MAX_DEPTH) * valid
    b2 = b.reshape(1, _COUT).astype(jnp.float32)
    idxb = idxT.reshape(_K, _NBLK_SC, _B).transpose(1, 0, 2)

    y, base = _tc_phase(x.astype(jnp.float32), deltaT, W.astype(jnp.float32),
                        b2)
    ytab = y.reshape(_NBLK_TC * _K * _BN, _C)

    out = _sc_gather_sum(ytab, base, idxb)
    return out[:_N]


# speedup vs baseline: 1.7123x; 1.0157x over previous
import functools

import jax
import jax.numpy as jnp
from jax import lax
from jax.experimental import pallas as pl
from jax.experimental.pallas import tpu as pltpu
from jax.experimental.pallas import tpu_sc as plsc

_N = 50000
_C = 128
_COUT = 128
_K = 9
_MAX_DEPTH = 8

_BN = 2048
_BN_LOG2 = 11
_NBLK_TC = (_N + _BN - 1) // _BN
_NPAD = _BN * _NBLK_TC
_B = 128
_NBLK_SC = _NPAD // _B
_NW = 32
_TMAX = (_NBLK_SC + _NW - 1) // _NW
_ZROW = _N


def _tc_body(x_ref, deltaT_ref, w_ref, b_ref, y_ref, base_ref):
    i = pl.program_id(0)
    row = i * _BN + lax.broadcasted_iota(jnp.int32, (_BN, 1), 0)
    xb = jnp.where(row < _N, x_ref[...], 0.0)
    for k in range(_K):
        wk = w_ref[k * (_C + 1):k * (_C + 1) + _C, :]
        y_ref[0, k * _BN:(k + 1) * _BN, :] = jnp.dot(
            xb, wk, preferred_element_type=jnp.float32)
    deltaT = deltaT_ref[...]
    wd = jnp.concatenate(
        [w_ref[k * (_C + 1) + _C:k * (_C + 1) + _C + 1, :] for k in range(_K)],
        axis=0)
    base = lax.dot_general(deltaT, wd, (((0,), (0,)), ((), ())),
                           preferred_element_type=jnp.float32)
    base_ref[...] = base + b_ref[...]


def _tc_phase(x, deltaT, W, b2):
    return pl.pallas_call(
        _tc_body,
        grid=(_NBLK_TC,),
        in_specs=[
            pl.BlockSpec((_BN, _C), lambda i: (jnp.minimum(i, _N // _BN), 0)),
            pl.BlockSpec((_K, _BN), lambda i: (0, i)),
            pl.BlockSpec((_K * (_C + 1), _COUT), lambda i: (0, 0)),
            pl.BlockSpec((1, _COUT), lambda i: (0, 0)),
        ],
        out_specs=[
            pl.BlockSpec((1, _K * _BN, _C), lambda i: (i, 0, 0)),
            pl.BlockSpec((_BN, _COUT), lambda i: (i, 0)),
        ],
        out_shape=[
            jax.ShapeDtypeStruct((_NBLK_TC, _K * _BN, _C), jnp.float32),
            jax.ShapeDtypeStruct((_NPAD, _COUT), jnp.float32),
        ],
    )(x, deltaT, W, b2)


@functools.partial(
    pl.kernel,
    out_type=jax.ShapeDtypeStruct((_NPAD, _COUT), jnp.float32),
    mesh=plsc.VectorSubcoreMesh(core_axis_name="c", subcore_axis_name="s"),
    scratch_types=[
        pltpu.VMEM((_K, _B), jnp.int32),
        pltpu.VMEM((_K, _B), jnp.int32),
        pltpu.VMEM((_B, _COUT), jnp.float32),
        pltpu.SemaphoreType.DMA,
    ],
    compiler_params=pltpu.CompilerParams(use_tc_tiling_on_sc=True),
)
def _sc_gather_sum(ytab, base_hbm, idxb_hbm, out_hbm, idxv, jv, acc, sem):
    cid = lax.axis_index("c")
    sid = lax.axis_index("s")
    wid = sid * 2 + cid

    def block_body(t, carry):
        blk = wid + _NW * t

        @pl.when(blk < _NBLK_SC)
        def _():
            off = blk * _B
            pltpu.sync_copy(idxb_hbm.at[blk], idxv)
            pltpu.sync_copy(base_hbm.at[pl.ds(off, _B)], acc)
            for k in range(_K):
                for u in range(_B // 16):
                    v = idxv[k, pl.ds(u * 16, 16)]
                    zrow = _ZROW + u * 16 + lax.iota(jnp.int32, 16)
                    vv = jnp.where(v >= 0, v, zrow)
                    jv[k, pl.ds(u * 16, 16)] = (
                        lax.shift_right_logical(vv, _BN_LOG2) * (_K * _BN)
                        + k * _BN + lax.bitwise_and(vv, _BN - 1))
            cps = [
                pltpu.async_copy(ytab.at[jv.at[k]], acc, sem, add=True)
                for k in range(_K)
            ]
            for cp in cps:
                cp.wait()
            pltpu.sync_copy(acc, out_hbm.at[pl.ds(off, _B)])

        return carry

    lax.fori_loop(0, _TMAX, block_body, 0)


def kernel(x, neigh_idx, neigh_depth, depth, W, b):
    idxT = jnp.pad(neigh_idx.astype(jnp.int32).T, ((0, 0), (0, _NPAD - _N)),
                   constant_values=-1)
    valid = (idxT >= 0).astype(jnp.float32)
    ndT = jnp.pad(neigh_depth.astype(jnp.float32).T,
                  ((0, 0), (0, _NPAD - _N)))
    deltaT = (jnp.asarray(depth, jnp.float32) - ndT) * (1.0 / _MAX_DEPTH) * valid
    b2 = b.reshape(1, _COUT).astype(jnp.float32)
    idxb = idxT.reshape(_K, _NBLK_SC, _B).transpose(1, 0, 2)

    y, base = _tc_phase(x.astype(jnp.float32), deltaT, W.astype(jnp.float32),
                        b2)
    ytab = y.reshape(_NBLK_TC * _K * _BN, _C)

    out = _sc_gather_sum(ytab, base, idxb)
    return out[:_N]
